# Initial kernel scaffold; baseline (speedup 1.0000x reference)
import jax
import jax.numpy as jnp
from jax.experimental import pallas as pl


def kernel(idx, id_table):
    raise NotImplementedError("write your pallas kernel here")


# speedup vs baseline: 10.8422x; 10.8422x over previous
import functools

import jax
import jax.numpy as jnp
from jax import lax
from jax.experimental import pallas as pl
from jax.experimental.pallas import tpu as pltpu
from jax.experimental.pallas import tpu_sc as plsc

_G = 128
_NB = 200
_R = 4


@functools.cache
def _build(n_idx_rows, d, n_table_rows):
    mesh = plsc.VectorSubcoreMesh(core_axis_name="c", subcore_axis_name="s")
    n_workers = 32
    rows_per_w = n_idx_rows // n_workers
    n_blocks = rows_per_w // _NB

    @functools.partial(
        pl.kernel,
        out_type=jax.ShapeDtypeStruct((n_idx_rows * _G, d), jnp.float32),
        mesh=mesh,
        scratch_types=[
            pltpu.VMEM((_NB, _G), jnp.int32),
            pltpu.VMEM((_R, _G, d), jnp.float32),
            pltpu.SemaphoreType.DMA,
            pltpu.SemaphoreType.DMA,
            pltpu.SemaphoreType.DMA,
        ],
    )
    def body(table_hbm, idx_hbm, out_hbm, idx_v, rows_v, isem, gsem, osem):
        wid = lax.axis_index("s") * 2 + lax.axis_index("c")
        wrow = wid * rows_per_w

        def do_block(ib, _):
            row0 = wrow + ib * _NB
            cp = pltpu.make_async_copy(
                idx_hbm.at[pl.ds(row0, _NB)], idx_v, isem)
            cp.start()
            cp.wait()

            def do_group(g, _):
                j0 = g * _R
                for r in range(_R):
                    pltpu.make_async_copy(
                        table_hbm.at[idx_v.at[j0 + r]],
                        rows_v.at[r], gsem).start()
                for r in range(_R):
                    pltpu.make_async_copy(
                        table_hbm.at[idx_v.at[j0 + r]],
                        rows_v.at[r], gsem).wait()
                    pltpu.make_async_copy(
                        rows_v.at[r],
                        out_hbm.at[pl.ds((row0 + j0 + r) * _G, _G)],
                        osem).start()
                for r in range(_R):
                    pltpu.make_async_copy(
                        rows_v.at[r],
                        out_hbm.at[pl.ds((row0 + j0 + r) * _G, _G)],
                        osem).wait()
                return ()

            lax.fori_loop(0, _NB // _R, do_group, (), unroll=False)
            return ()

        lax.fori_loop(0, n_blocks, do_block, (), unroll=False)

    return body


def kernel(idx, id_table):
    b, k = idx.shape
    n_table_rows, d = id_table.shape
    flat = idx.reshape(-1).astype(jnp.int32)
    n = flat.shape[0]
    idx2d = flat.reshape(n // _G, _G)
    out = _build(n // _G, d, n_table_rows)(id_table, idx2d)
    return out.reshape(b, k, d)
